# Initial kernel scaffold; baseline (speedup 1.0000x reference)
import jax
import jax.numpy as jnp
from jax.experimental import pallas as pl


def kernel(x, adj_indices, adj_values, weight, bias):
    raise NotImplementedError("write your pallas kernel here")


# speedup vs baseline: 6.3802x; 6.3802x over previous
import functools

import jax
import jax.numpy as jnp
from jax import lax
from jax.experimental import pallas as pl
from jax.experimental.pallas import tpu as pltpu
from jax.experimental.pallas import tpu_sc as plsc

N = 10000
E = 320000
D = 128

NC = 2
NS = 16
NW = NC * NS
EPW = E // NW
K = 80
NCHUNK = EPW // K
N_PAD = 10240
RPT = N_PAD // NS

MM_BLOCK = 1000


def _mm_body(x_ref, w_ref, o_ref):
    o_ref[...] = jnp.dot(x_ref[...], w_ref[...],
                         preferred_element_type=jnp.float32)


def _combine_body(p0_ref, p1_ref, b_ref, o_ref):
    o_ref[...] = p0_ref[...] + p1_ref[...] + b_ref[...]


def _sc_body(col_hbm, row_hbm, val_hbm, sup_hbm, zero_hbm, out_hbm,
             colv, rowv, valv, gath, acc):
    c = lax.axis_index("c")
    s = lax.axis_index("s")
    wid = s * NC + c

    pltpu.sync_copy(zero_hbm, acc.at[pl.ds(s * RPT, RPT)])
    pltpu.sync_copy(col_hbm.at[wid], colv)
    pltpu.sync_copy(row_hbm.at[wid], rowv)
    pltpu.sync_copy(val_hbm.at[wid], valv)
    plsc.subcore_barrier()

    def chunk(g, carry):
        pltpu.sync_copy(sup_hbm.at[colv.at[pl.ds(g * K, K)]], gath)

        def group(eg, carry2):
            vgroup = valv[pl.ds(g * K + eg * 16, 16)]
            for e16 in range(16):
                vb = jnp.full((16,), vgroup[e16])
                e = eg * 16 + e16
                for f in range(D // 16):
                    gath[e, pl.ds(f * 16, 16)] = (
                        gath[e, pl.ds(f * 16, 16)] * vb)
            return carry2

        lax.fori_loop(0, K // 16, group, 0)

        pltpu.sync_copy(gath, acc.at[rowv.at[g]], add=True)
        return carry

    lax.fori_loop(0, NCHUNK, chunk, 0)
    plsc.subcore_barrier()

    pltpu.sync_copy(acc.at[pl.ds(s * RPT, RPT)],
                    out_hbm.at[c, pl.ds(s * RPT, RPT)])


def kernel(x, adj_indices, adj_values, weight, bias):
    adj = adj_indices.astype(jnp.int32)
    row3 = adj[0].reshape(NW, NCHUNK, K)
    col3 = adj[1].reshape(NW, EPW)
    val2 = adj_values.reshape(NW, EPW)
    zeros = jnp.zeros((RPT, D), jnp.float32)

    support = pl.pallas_call(
        _mm_body,
        grid=(N // MM_BLOCK,),
        in_specs=[
            pl.BlockSpec((MM_BLOCK, D), lambda i: (i, 0)),
            pl.BlockSpec((D, D), lambda i: (0, 0)),
        ],
        out_specs=pl.BlockSpec((MM_BLOCK, D), lambda i: (i, 0)),
        out_shape=jax.ShapeDtypeStruct((N, D), jnp.float32),
    )(x, weight)

    sc = functools.partial(
        pl.kernel,
        mesh=plsc.VectorSubcoreMesh(core_axis_name="c", subcore_axis_name="s"),
        out_type=jax.ShapeDtypeStruct((NC, N_PAD, D), jnp.float32),
        scratch_types=[
            pltpu.VMEM((EPW,), jnp.int32),
            pltpu.VMEM((NCHUNK, K), jnp.int32),
            pltpu.VMEM((EPW,), jnp.float32),
            pltpu.VMEM((K, D), jnp.float32),
            pltpu.VMEM_SHARED((N_PAD, D), jnp.float32),
        ],
    )(_sc_body)
    partials = sc(col3, row3, val2, support, zeros)

    out = pl.pallas_call(
        _combine_body,
        grid=(N // MM_BLOCK,),
        in_specs=[
            pl.BlockSpec((MM_BLOCK, D), lambda i: (i, 0)),
            pl.BlockSpec((MM_BLOCK, D), lambda i: (i, 0)),
            pl.BlockSpec((1, D), lambda i: (0, 0)),
        ],
        out_specs=pl.BlockSpec((MM_BLOCK, D), lambda i: (i, 0)),
        out_shape=jax.ShapeDtypeStruct((N, D), jnp.float32),
    )(partials[0], partials[1], bias.reshape(1, D))
    return out


# speedup vs baseline: 10.1091x; 1.5845x over previous
import functools

import jax
import jax.numpy as jnp
from jax import lax
from jax.experimental import pallas as pl
from jax.experimental.pallas import tpu as pltpu
from jax.experimental.pallas import tpu_sc as plsc

N = 10000
E = 320000
D = 128

NC = 2
NS = 16
NW = NC * NS
EPW = E // NW
K = 80
NCHUNK = EPW // K
N_PAD = 10240
RPT = N_PAD // NS

MM_BLOCK = 1000


def _mm_body(x_ref, w_ref, o_ref):
    o_ref[...] = jnp.dot(x_ref[...], w_ref[...],
                         preferred_element_type=jnp.float32)


def _combine_body(p0_ref, p1_ref, b_ref, o_ref):
    o_ref[...] = p0_ref[...] + p1_ref[...] + b_ref[...]


def _sc_body(col_hbm, row_hbm, val_hbm, sup_hbm, zero_hbm, out_hbm,
             rowv, cb0, cb1, vb0, vb1, gath0, gath1,
             semc0, semc1, semv0, semv1, semg0, semg1, sems0, sems1,
             acc_ref):
    c = lax.axis_index("c")
    s = lax.axis_index("s")
    wid = s * NC + c

    cb = (cb0, cb1)
    vb = (vb0, vb1)
    gath = (gath0, gath1)
    semc = (semc0, semc1)
    semv = (semv0, semv1)
    semg = (semg0, semg1)
    sems = (sems0, sems1)

    pltpu.sync_copy(zero_hbm, acc_ref.at[pl.ds(s * RPT, RPT)])
    pltpu.sync_copy(row_hbm.at[wid], rowv)
    plsc.subcore_barrier()

    def col_src(g):
        return col_hbm.at[pl.ds(wid * EPW + g * K, K)]

    def val_src(g):
        return val_hbm.at[pl.ds(wid * EPW + g * K, K)]

    pltpu.sync_copy(col_src(0), cb[0])
    pltpu.async_copy(col_src(1), cb[1], semc[1])
    pltpu.async_copy(val_src(0), vb[0], semv[0])
    pltpu.async_copy(sup_hbm.at[cb[0]], gath[0], semg[0])

    def emit_iter(g, b):
        nb = 1 - b

        @pl.when(g >= 1)
        def _():
            pltpu.make_async_copy(
                gath[nb], acc_ref.at[rowv.at[g - 1]], sems[nb]).wait()

        @pl.when(g + 1 < NCHUNK)
        def _():
            pltpu.make_async_copy(col_src(g + 1), cb[nb], semc[nb]).wait()
            pltpu.async_copy(sup_hbm.at[cb[nb]], gath[nb], semg[nb])
            pltpu.async_copy(val_src(g + 1), vb[nb], semv[nb])

        pltpu.make_async_copy(sup_hbm.at[cb[b]], gath[b], semg[b]).wait()

        @pl.when(g + 2 < NCHUNK)
        def _():
            pltpu.async_copy(col_src(g + 2), cb[b], semc[b])

        pltpu.make_async_copy(val_src(g), vb[b], semv[b]).wait()

        def group(eg, carry2):
            vgroup = vb[b][pl.ds(eg * 16, 16)]
            for e16 in range(16):
                vsc = jnp.full((16,), vgroup[e16])
                e = eg * 16 + e16
                for f in range(D // 16):
                    gath[b][e, pl.ds(f * 16, 16)] = (
                        gath[b][e, pl.ds(f * 16, 16)] * vsc)
            return carry2

        lax.fori_loop(0, K // 16, group, 0)

        pltpu.async_copy(gath[b], acc_ref.at[rowv.at[g]], sems[b], add=True)

    def pair(i, carry):
        emit_iter(2 * i, 0)
        emit_iter(2 * i + 1, 1)
        return carry

    lax.fori_loop(0, NCHUNK // 2, pair, 0)
    emit_iter(NCHUNK - 1, 0)
    pltpu.make_async_copy(
        gath[0], acc_ref.at[rowv.at[NCHUNK - 1]], sems[0]).wait()
    plsc.subcore_barrier()

    pltpu.sync_copy(acc_ref.at[pl.ds(s * RPT, RPT)],
                    out_hbm.at[c, pl.ds(s * RPT, RPT)])


def kernel(x, adj_indices, adj_values, weight, bias):
    adj = adj_indices.astype(jnp.int32)
    row3 = adj[0].reshape(NW, NCHUNK, K)
    col1 = adj[1]
    val1 = adj_values
    zeros = jnp.zeros((RPT, D), jnp.float32)

    support = pl.pallas_call(
        _mm_body,
        grid=(N // MM_BLOCK,),
        in_specs=[
            pl.BlockSpec((MM_BLOCK, D), lambda i: (i, 0)),
            pl.BlockSpec((D, D), lambda i: (0, 0)),
        ],
        out_specs=pl.BlockSpec((MM_BLOCK, D), lambda i: (i, 0)),
        out_shape=jax.ShapeDtypeStruct((N, D), jnp.float32),
    )(x, weight)

    sc = functools.partial(
        pl.kernel,
        mesh=plsc.VectorSubcoreMesh(core_axis_name="c", subcore_axis_name="s"),
        out_type=jax.ShapeDtypeStruct((NC, N_PAD, D), jnp.float32),
        scratch_types=[
            pltpu.VMEM((NCHUNK, K), jnp.int32),
            pltpu.VMEM((K,), jnp.int32),
            pltpu.VMEM((K,), jnp.int32),
            pltpu.VMEM((K,), jnp.float32),
            pltpu.VMEM((K,), jnp.float32),
            pltpu.VMEM((K, D), jnp.float32),
            pltpu.VMEM((K, D), jnp.float32),
            pltpu.SemaphoreType.DMA,
            pltpu.SemaphoreType.DMA,
            pltpu.SemaphoreType.DMA,
            pltpu.SemaphoreType.DMA,
            pltpu.SemaphoreType.DMA,
            pltpu.SemaphoreType.DMA,
            pltpu.SemaphoreType.DMA,
            pltpu.SemaphoreType.DMA,
            pltpu.VMEM_SHARED((N_PAD, D), jnp.float32),
        ],
    )(_sc_body)
    partials = sc(col1, row3, val1, support, zeros)

    out = pl.pallas_call(
        _combine_body,
        grid=(N // MM_BLOCK,),
        in_specs=[
            pl.BlockSpec((MM_BLOCK, D), lambda i: (i, 0)),
            pl.BlockSpec((MM_BLOCK, D), lambda i: (i, 0)),
            pl.BlockSpec((1, D), lambda i: (0, 0)),
        ],
        out_specs=pl.BlockSpec((MM_BLOCK, D), lambda i: (i, 0)),
        out_shape=jax.ShapeDtypeStruct((N, D), jnp.float32),
    )(partials[0], partials[1], bias.reshape(1, D))
    return out


# speedup vs baseline: 10.6627x; 1.0548x over previous
import functools

import jax
import jax.numpy as jnp
from jax import lax
from jax.experimental import pallas as pl
from jax.experimental.pallas import tpu as pltpu
from jax.experimental.pallas import tpu_sc as plsc

N = 10000
E = 320000
D = 128

NC = 2
NS = 16
NW = NC * NS
EPW = E // NW
K = 80
NCHUNK = EPW // K
N_PAD = 10240
RPT = N_PAD // NS

MM_BLOCK = 1000


def _mm_body(x_ref, w_ref, o_ref):
    o_ref[...] = jnp.dot(x_ref[...], w_ref[...],
                         preferred_element_type=jnp.float32)


def _combine_body(p0_ref, p1_ref, b_ref, o_ref):
    o_ref[...] = p0_ref[...] + p1_ref[...] + b_ref[...]


def _sc_body(col_hbm, row_hbm, val_hbm, sup_hbm, zero_hbm, out_hbm,
             rowv, cb0, cb1, cb2, vb0, vb1, vb2, gath0, gath1, gath2,
             semc0, semc1, semc2, semv0, semv1, semv2,
             semg0, semg1, semg2, sems0, sems1, sems2,
             acc_ref):
    c = lax.axis_index("c")
    s = lax.axis_index("s")
    wid = s * NC + c

    cb = (cb0, cb1, cb2)
    vb = (vb0, vb1, vb2)
    gath = (gath0, gath1, gath2)
    semc = (semc0, semc1, semc2)
    semv = (semv0, semv1, semv2)
    semg = (semg0, semg1, semg2)
    sems = (sems0, sems1, sems2)

    pltpu.sync_copy(zero_hbm, acc_ref.at[pl.ds(s * RPT, RPT)])
    pltpu.sync_copy(row_hbm.at[wid], rowv)
    plsc.subcore_barrier()

    def col_src(g):
        return col_hbm.at[pl.ds(wid * EPW + g * K, K)]

    def val_src(g):
        return val_hbm.at[pl.ds(wid * EPW + g * K, K)]

    pltpu.sync_copy(col_src(0), cb[0])
    pltpu.async_copy(col_src(1), cb[1], semc[1])
    pltpu.async_copy(val_src(0), vb[0], semv[0])
    pltpu.async_copy(sup_hbm.at[cb[0]], gath[0], semg[0])

    def emit_iter(g, b):
        b1 = (b + 1) % 3
        b2 = (b + 2) % 3

        @pl.when(g >= 2)
        def _():
            pltpu.make_async_copy(
                gath[b1], acc_ref.at[rowv.at[g - 2]], sems[b1]).wait()

        @pl.when(g + 1 < NCHUNK)
        def _():
            pltpu.make_async_copy(col_src(g + 1), cb[b1], semc[b1]).wait()
            pltpu.async_copy(sup_hbm.at[cb[b1]], gath[b1], semg[b1])
            pltpu.async_copy(val_src(g + 1), vb[b1], semv[b1])

        pltpu.make_async_copy(sup_hbm.at[cb[b]], gath[b], semg[b]).wait()

        @pl.when(g + 2 < NCHUNK)
        def _():
            pltpu.async_copy(col_src(g + 2), cb[b2], semc[b2])

        pltpu.make_async_copy(val_src(g), vb[b], semv[b]).wait()

        def group(eg, carry2):
            vgroup = vb[b][pl.ds(eg * 16, 16)]
            for e16 in range(16):
                vsc = jnp.full((16,), vgroup[e16])
                e = eg * 16 + e16
                for f in range(D // 16):
                    gath[b][e, pl.ds(f * 16, 16)] = (
                        gath[b][e, pl.ds(f * 16, 16)] * vsc)
            return carry2

        lax.fori_loop(0, K // 16, group, 0)

        pltpu.async_copy(gath[b], acc_ref.at[rowv.at[g]], sems[b], add=True)

    def tri(i, carry):
        emit_iter(3 * i, 0)
        emit_iter(3 * i + 1, 1)
        emit_iter(3 * i + 2, 2)
        return carry

    lax.fori_loop(0, NCHUNK // 3, tri, 0)
    emit_iter(NCHUNK - 2, 0)
    emit_iter(NCHUNK - 1, 1)
    pltpu.make_async_copy(
        gath[0], acc_ref.at[rowv.at[NCHUNK - 2]], sems[0]).wait()
    pltpu.make_async_copy(
        gath[1], acc_ref.at[rowv.at[NCHUNK - 1]], sems[1]).wait()
    plsc.subcore_barrier()

    pltpu.sync_copy(acc_ref.at[pl.ds(s * RPT, RPT)],
                    out_hbm.at[c, pl.ds(s * RPT, RPT)])


def kernel(x, adj_indices, adj_values, weight, bias):
    adj = adj_indices.astype(jnp.int32)
    row3 = adj[0].reshape(NW, NCHUNK, K)
    col1 = adj[1]
    val1 = adj_values
    zeros = jnp.zeros((RPT, D), jnp.float32)

    support = pl.pallas_call(
        _mm_body,
        grid=(N // MM_BLOCK,),
        in_specs=[
            pl.BlockSpec((MM_BLOCK, D), lambda i: (i, 0)),
            pl.BlockSpec((D, D), lambda i: (0, 0)),
        ],
        out_specs=pl.BlockSpec((MM_BLOCK, D), lambda i: (i, 0)),
        out_shape=jax.ShapeDtypeStruct((N, D), jnp.float32),
    )(x, weight)

    sc = functools.partial(
        pl.kernel,
        mesh=plsc.VectorSubcoreMesh(core_axis_name="c", subcore_axis_name="s"),
        out_type=jax.ShapeDtypeStruct((NC, N_PAD, D), jnp.float32),
        scratch_types=[
            pltpu.VMEM((NCHUNK, K), jnp.int32),
            pltpu.VMEM((K,), jnp.int32),
            pltpu.VMEM((K,), jnp.int32),
            pltpu.VMEM((K,), jnp.int32),
            pltpu.VMEM((K,), jnp.float32),
            pltpu.VMEM((K,), jnp.float32),
            pltpu.VMEM((K,), jnp.float32),
            pltpu.VMEM((K, D), jnp.float32),
            pltpu.VMEM((K, D), jnp.float32),
            pltpu.VMEM((K, D), jnp.float32),
            pltpu.SemaphoreType.DMA,
            pltpu.SemaphoreType.DMA,
            pltpu.SemaphoreType.DMA,
            pltpu.SemaphoreType.DMA,
            pltpu.SemaphoreType.DMA,
            pltpu.SemaphoreType.DMA,
            pltpu.SemaphoreType.DMA,
            pltpu.SemaphoreType.DMA,
            pltpu.SemaphoreType.DMA,
            pltpu.SemaphoreType.DMA,
            pltpu.SemaphoreType.DMA,
            pltpu.SemaphoreType.DMA,
            pltpu.VMEM_SHARED((N_PAD, D), jnp.float32),
        ],
    )(_sc_body)
    partials = sc(col1, row3, val1, support, zeros)

    out = pl.pallas_call(
        _combine_body,
        grid=(N // MM_BLOCK,),
        in_specs=[
            pl.BlockSpec((MM_BLOCK, D), lambda i: (i, 0)),
            pl.BlockSpec((MM_BLOCK, D), lambda i: (i, 0)),
            pl.BlockSpec((1, D), lambda i: (0, 0)),
        ],
        out_specs=pl.BlockSpec((MM_BLOCK, D), lambda i: (i, 0)),
        out_shape=jax.ShapeDtypeStruct((N, D), jnp.float32),
    )(partials[0], partials[1], bias.reshape(1, D))
    return out


# speedup vs baseline: 11.5578x; 1.0839x over previous
import functools

import jax
import jax.numpy as jnp
from jax import lax
from jax.experimental import pallas as pl
from jax.experimental.pallas import tpu as pltpu
from jax.experimental.pallas import tpu_sc as plsc

N = 10000
E = 320000
D = 128

NC = 2
NS = 16
NW = NC * NS
EPW = E // NW
K = 80
NCHUNK = EPW // K
NBUF = 4
N_PAD = 10240
RPT = N_PAD // NS

MM_BLOCK = 1000


def _mm_body(x_ref, w_ref, o_ref):
    o_ref[...] = jnp.dot(x_ref[...], w_ref[...],
                         preferred_element_type=jnp.float32)


def _combine_body(p0_ref, p1_ref, b_ref, o_ref):
    o_ref[...] = p0_ref[...] + p1_ref[...] + b_ref[...]


def _sc_body(col_hbm, row_hbm, val_hbm, sup_hbm, zero_hbm, out_hbm,
             cb0, cb1, cb2, cb3, rb0, rb1, rb2, rb3, vb0, vb1, vb2, vb3,
             gath0, gath1, gath2, gath3,
             semc0, semc1, semc2, semc3, semr0, semr1, semr2, semr3,
             semv0, semv1, semv2, semv3, semg0, semg1, semg2, semg3,
             sems0, sems1, sems2, sems3,
             acc_ref):
    c = lax.axis_index("c")
    s = lax.axis_index("s")
    wid = s * NC + c

    cb = (cb0, cb1, cb2, cb3)
    rb = (rb0, rb1, rb2, rb3)
    vb = (vb0, vb1, vb2, vb3)
    gath = (gath0, gath1, gath2, gath3)
    semc = (semc0, semc1, semc2, semc3)
    semr = (semr0, semr1, semr2, semr3)
    semv = (semv0, semv1, semv2, semv3)
    semg = (semg0, semg1, semg2, semg3)
    sems = (sems0, sems1, sems2, sems3)

    def col_src(g):
        return col_hbm.at[pl.ds(wid * EPW + g * K, K)]

    def row_src(g):
        return row_hbm.at[pl.ds(wid * EPW + g * K, K)]

    def val_src(g):
        return val_hbm.at[pl.ds(wid * EPW + g * K, K)]

    pltpu.sync_copy(zero_hbm, acc_ref.at[pl.ds(s * RPT, RPT)])
    plsc.subcore_barrier()

    pltpu.sync_copy(col_src(0), cb[0])
    pltpu.async_copy(col_src(1), cb[1], semc[1])
    pltpu.async_copy(col_src(2), cb[2], semc[2])
    pltpu.async_copy(val_src(0), vb[0], semv[0])
    pltpu.async_copy(val_src(1), vb[1], semv[1])
    pltpu.async_copy(row_src(0), rb[0], semr[0])
    pltpu.async_copy(row_src(1), rb[1], semr[1])
    pltpu.async_copy(sup_hbm.at[cb[0]], gath[0], semg[0])
    pltpu.make_async_copy(col_src(1), cb[1], semc[1]).wait()
    pltpu.async_copy(sup_hbm.at[cb[1]], gath[1], semg[1])

    def emit_iter(g, b):
        b2 = (b + 2) % NBUF
        b3 = (b + 3) % NBUF

        @pl.when(g >= 2)
        def _():
            pltpu.make_async_copy(
                gath[b2], acc_ref.at[rb[b2]], sems[b2]).wait()

        @pl.when(g + 2 < NCHUNK)
        def _():
            pltpu.make_async_copy(col_src(g + 2), cb[b2], semc[b2]).wait()
            pltpu.async_copy(sup_hbm.at[cb[b2]], gath[b2], semg[b2])
            pltpu.async_copy(val_src(g + 2), vb[b2], semv[b2])
            pltpu.async_copy(row_src(g + 2), rb[b2], semr[b2])

        @pl.when(g + 3 < NCHUNK)
        def _():
            pltpu.async_copy(col_src(g + 3), cb[b3], semc[b3])

        pltpu.make_async_copy(sup_hbm.at[cb[b]], gath[b], semg[b]).wait()
        pltpu.make_async_copy(val_src(g), vb[b], semv[b]).wait()

        def group(eg, carry2):
            vgroup = vb[b][pl.ds(eg * 16, 16)]
            for e16 in range(16):
                vsc = jnp.full((16,), vgroup[e16])
                e = eg * 16 + e16
                for f in range(D // 16):
                    gath[b][e, pl.ds(f * 16, 16)] = (
                        gath[b][e, pl.ds(f * 16, 16)] * vsc)
            return carry2

        lax.fori_loop(0, K // 16, group, 0)

        pltpu.make_async_copy(row_src(g), rb[b], semr[b]).wait()
        pltpu.async_copy(gath[b], acc_ref.at[rb[b]], sems[b], add=True)

    def quad(i, carry):
        emit_iter(4 * i, 0)
        emit_iter(4 * i + 1, 1)
        emit_iter(4 * i + 2, 2)
        emit_iter(4 * i + 3, 3)
        return carry

    lax.fori_loop(0, (NCHUNK - 1) // NBUF, quad, 0)
    emit_iter(NCHUNK - 1, 0)
    pltpu.make_async_copy(
        gath[3], acc_ref.at[rb[3]], sems[3]).wait()
    pltpu.make_async_copy(
        gath[0], acc_ref.at[rb[0]], sems[0]).wait()
    plsc.subcore_barrier()

    pltpu.sync_copy(acc_ref.at[pl.ds(s * RPT, RPT)],
                    out_hbm.at[c, pl.ds(s * RPT, RPT)])


def kernel(x, adj_indices, adj_values, weight, bias):
    adj = adj_indices.astype(jnp.int32)
    row1 = adj[0]
    col1 = adj[1]
    val1 = adj_values
    zeros = jnp.zeros((RPT, D), jnp.float32)

    support = pl.pallas_call(
        _mm_body,
        grid=(N // MM_BLOCK,),
        in_specs=[
            pl.BlockSpec((MM_BLOCK, D), lambda i: (i, 0)),
            pl.BlockSpec((D, D), lambda i: (0, 0)),
        ],
        out_specs=pl.BlockSpec((MM_BLOCK, D), lambda i: (i, 0)),
        out_shape=jax.ShapeDtypeStruct((N, D), jnp.float32),
    )(x, weight)

    buf_types = []
    for dt in (jnp.int32, jnp.int32, jnp.float32):
        buf_types += [pltpu.VMEM((K,), dt)] * NBUF
    buf_types += [pltpu.VMEM((K, D), jnp.float32)] * NBUF
    sem_types = [pltpu.SemaphoreType.DMA] * (5 * NBUF)

    sc = functools.partial(
        pl.kernel,
        mesh=plsc.VectorSubcoreMesh(core_axis_name="c", subcore_axis_name="s"),
        out_type=jax.ShapeDtypeStruct((NC, N_PAD, D), jnp.float32),
        scratch_types=buf_types + sem_types + [
            pltpu.VMEM_SHARED((N_PAD, D), jnp.float32),
        ],
    )(_sc_body)
    partials = sc(col1, row1, val1, support, zeros)

    out = pl.pallas_call(
        _combine_body,
        grid=(N // MM_BLOCK,),
        in_specs=[
            pl.BlockSpec((MM_BLOCK, D), lambda i: (i, 0)),
            pl.BlockSpec((MM_BLOCK, D), lambda i: (i, 0)),
            pl.BlockSpec((1, D), lambda i: (0, 0)),
        ],
        out_specs=pl.BlockSpec((MM_BLOCK, D), lambda i: (i, 0)),
        out_shape=jax.ShapeDtypeStruct((N, D), jnp.float32),
    )(partials[0], partials[1], bias.reshape(1, D))
    return out


# speedup vs baseline: 12.0530x; 1.0428x over previous
import functools

import jax
import jax.numpy as jnp
from jax import lax
from jax.experimental import pallas as pl
from jax.experimental.pallas import tpu as pltpu
from jax.experimental.pallas import tpu_sc as plsc

N = 10000
E = 320000
D = 128

NC = 2
NS = 16
NW = NC * NS
EPW = E // NW
K = 80
NCHUNK = EPW // K
NBUF = 4
N_PAD = 10240
RPT = N_PAD // NS

MM_BLOCK = 1000


def _mm_body(x_ref, w_ref, o_ref):
    o_ref[...] = jnp.dot(x_ref[...], w_ref[...],
                         preferred_element_type=jnp.float32)


def _combine_body(p0_ref, p1_ref, b_ref, o_ref):
    o_ref[...] = p0_ref[...] + p1_ref[...] + b_ref[...]


def _sc_body(col_hbm, row_hbm, val_hbm, sup_hbm, zero_hbm, out_hbm,
             cb0, cb1, cb2, cb3, rb0, rb1, rb2, rb3, vb0, vb1, vb2, vb3,
             gath0, gath1, gath2, gath3,
             semc0, semc1, semc2, semc3, semr0, semr1, semr2, semr3,
             semv0, semv1, semv2, semv3, semg0, semg1, semg2, semg3,
             sems0, sems1, sems2, sems3,
             acc_ref):
    c = lax.axis_index("c")
    s = lax.axis_index("s")
    wid = s * NC + c

    cb = (cb0, cb1, cb2, cb3)
    rb = (rb0, rb1, rb2, rb3)
    vb = (vb0, vb1, vb2, vb3)
    gath = (gath0, gath1, gath2, gath3)
    semc = (semc0, semc1, semc2, semc3)
    semr = (semr0, semr1, semr2, semr3)
    semv = (semv0, semv1, semv2, semv3)
    semg = (semg0, semg1, semg2, semg3)
    sems = (sems0, sems1, sems2, sems3)

    def col_src(g):
        return col_hbm.at[pl.ds(wid * EPW + g * K, K)]

    def row_src(g):
        return row_hbm.at[pl.ds(wid * EPW + g * K, K)]

    def val_src(g):
        return val_hbm.at[pl.ds(wid * EPW + g * K, K)]

    pltpu.sync_copy(zero_hbm, acc_ref.at[pl.ds(s * RPT, RPT)])
    plsc.subcore_barrier()

    pltpu.sync_copy(col_src(0), cb[0])
    pltpu.async_copy(col_src(1), cb[1], semc[1])
    pltpu.async_copy(col_src(2), cb[2], semc[2])
    pltpu.async_copy(val_src(0), vb[0], semv[0])
    pltpu.async_copy(val_src(1), vb[1], semv[1])
    pltpu.async_copy(row_src(0), rb[0], semr[0])
    pltpu.async_copy(row_src(1), rb[1], semr[1])
    pltpu.async_copy(sup_hbm.at[cb[0]], gath[0], semg[0])
    pltpu.make_async_copy(col_src(1), cb[1], semc[1]).wait()
    pltpu.async_copy(sup_hbm.at[cb[1]], gath[1], semg[1])

    def emit_iter(g, b):
        b2 = (b + 2) % NBUF
        b3 = (b + 3) % NBUF

        @pl.when(g >= 2)
        def _():
            pltpu.make_async_copy(
                gath[b2], acc_ref.at[rb[b2]], sems[b2]).wait()

        @pl.when(g + 2 < NCHUNK)
        def _():
            pltpu.make_async_copy(col_src(g + 2), cb[b2], semc[b2]).wait()
            pltpu.async_copy(sup_hbm.at[cb[b2]], gath[b2], semg[b2])
            pltpu.async_copy(val_src(g + 2), vb[b2], semv[b2])
            pltpu.async_copy(row_src(g + 2), rb[b2], semr[b2])

        @pl.when(g + 3 < NCHUNK)
        def _():
            pltpu.async_copy(col_src(g + 3), cb[b3], semc[b3])

        pltpu.make_async_copy(sup_hbm.at[cb[b]], gath[b], semg[b]).wait()
        pltpu.make_async_copy(val_src(g), vb[b], semv[b]).wait()

        def group(eg, carry2):
            vgroup = vb[b][pl.ds(eg * 16, 16)]
            for e16 in range(16):
                vsc = jnp.full((16,), vgroup[e16])
                e = eg * 16 + e16
                for f in range(D // 16):
                    gath[b][e, pl.ds(f * 16, 16)] = (
                        gath[b][e, pl.ds(f * 16, 16)] * vsc)
            return carry2

        lax.fori_loop(0, K // 16, group, 0)

        pltpu.make_async_copy(row_src(g), rb[b], semr[b]).wait()
        pltpu.async_copy(gath[b], acc_ref.at[rb[b]], sems[b], add=True)

    def quad(i, carry):
        emit_iter(4 * i, 0)
        emit_iter(4 * i + 1, 1)
        emit_iter(4 * i + 2, 2)
        emit_iter(4 * i + 3, 3)
        return carry

    lax.fori_loop(0, (NCHUNK - 1) // NBUF, quad, 0)
    emit_iter(NCHUNK - 1, 0)
    pltpu.make_async_copy(
        gath[3], acc_ref.at[rb[3]], sems[3]).wait()
    pltpu.make_async_copy(
        gath[0], acc_ref.at[rb[0]], sems[0]).wait()
    plsc.subcore_barrier()

    pltpu.sync_copy(acc_ref.at[pl.ds(s * RPT, RPT)],
                    out_hbm.at[c, pl.ds(s * RPT, RPT)])


def kernel(x, adj_indices, adj_values, weight, bias):
    adj = adj_indices.astype(jnp.int32)
    row1 = adj[0]
    col1 = adj[1]
    val1 = adj_values
    zeros = jnp.zeros((RPT, D), jnp.float32)

    support = pl.pallas_call(
        _mm_body,
        out_shape=jax.ShapeDtypeStruct((N, D), jnp.float32),
    )(x, weight)

    buf_types = []
    for dt in (jnp.int32, jnp.int32, jnp.float32):
        buf_types += [pltpu.VMEM((K,), dt)] * NBUF
    buf_types += [pltpu.VMEM((K, D), jnp.float32)] * NBUF
    sem_types = [pltpu.SemaphoreType.DMA] * (5 * NBUF)

    sc = functools.partial(
        pl.kernel,
        mesh=plsc.VectorSubcoreMesh(core_axis_name="c", subcore_axis_name="s"),
        out_type=jax.ShapeDtypeStruct((NC, N_PAD, D), jnp.float32),
        scratch_types=buf_types + sem_types + [
            pltpu.VMEM_SHARED((N_PAD, D), jnp.float32),
        ],
    )(_sc_body)
    partials = sc(col1, row1, val1, support, zeros)

    out = pl.pallas_call(
        _combine_body,
        out_shape=jax.ShapeDtypeStruct((N, D), jnp.float32),
    )(partials[0][:N], partials[1][:N], bias.reshape(1, D))
    return out
